# baseline (speedup 1.0000x reference)
import functools

import jax
import jax.numpy as jnp
from jax import lax
from jax.experimental import pallas as pl
from jax.experimental.pallas import tpu as pltpu
from jax.experimental.pallas import tpu_sc as plsc

B = 16384
NC = 2
NS = 16
NW = NC * NS
BPW = B // NW
L = 16


def _gather_mu_body(uid_hbm, iid_hbm, umu_hbm, imu_hbm,
                    gumu_out, gimu_out,
                    uid_v, iid_v, umu_v, imu_v, sem0, sem1):
    wid = lax.axis_index("s") * NC + lax.axis_index("c")
    base = wid * BPW
    pltpu.sync_copy(uid_hbm.at[pl.ds(base, BPW)], uid_v)
    pltpu.sync_copy(iid_hbm.at[pl.ds(base, BPW)], iid_v)
    c0 = pltpu.async_copy(umu_hbm.at[uid_v], umu_v, sem0)
    c1 = pltpu.async_copy(imu_hbm.at[iid_v], imu_v, sem1)
    c0.wait()
    c1.wait()
    pltpu.sync_copy(umu_v, gumu_out.at[pl.ds(base, BPW)])
    pltpu.sync_copy(imu_v, gimu_out.at[pl.ds(base, BPW)])


def _ups_math_body(uid_hbm, iid_hbm, mu_hbm, ups_hbm,
                   uups_hbm, iups_hbm, gumu_hbm, gimu_hbm,
                   mu_out, ups_out, alpha_out, beta_out,
                   uid_v, iid_v, mu_v, ups_v,
                   umu_v, imu_v, uups_v, iups_v,
                   omu_v, oups_v, oalpha_v, obeta_v,
                   sem0, sem1):
    wid = lax.axis_index("s") * NC + lax.axis_index("c")
    base = wid * BPW

    pltpu.sync_copy(uid_hbm.at[pl.ds(base, BPW)], uid_v)
    pltpu.sync_copy(iid_hbm.at[pl.ds(base, BPW)], iid_v)

    c0 = pltpu.async_copy(uups_hbm.at[uid_v], uups_v, sem0)
    c1 = pltpu.async_copy(iups_hbm.at[iid_v], iups_v, sem1)

    pltpu.sync_copy(mu_hbm.at[pl.ds(base, BPW)], mu_v)
    pltpu.sync_copy(ups_hbm.at[pl.ds(base, BPW)], ups_v)
    pltpu.sync_copy(gumu_hbm.at[pl.ds(base, BPW)], umu_v)
    pltpu.sync_copy(gimu_hbm.at[pl.ds(base, BPW)], imu_v)
    c0.wait()
    c1.wait()

    eps = jnp.float32(1e-6)
    eps_param = jnp.float32(0.01)
    half = jnp.float32(0.5)
    one = jnp.float32(1.0)
    hi = jnp.float32(15.0)

    for i in range(BPW // L):
        s = pl.ds(i * L, L)
        mu = mu_v[s]
        ups = ups_v[s]
        umu = umu_v[s]
        imu = imu_v[s]
        uups = uups_v[s]
        iups = iups_v[s]

        ups2 = jnp.minimum(jnp.maximum(ups * (uups * iups), eps), hi)
        prod = umu * imu
        mu2 = jnp.where(
            mu < prod,
            half * mu / jnp.maximum(prod, eps),
            half + half * (mu - prod) / jnp.maximum(one - prod, eps),
        )
        alpha = jnp.maximum(mu2 * ups2, eps_param)
        beta = jnp.maximum(ups2 - alpha, eps_param)

        omu_v[s] = mu2
        oups_v[s] = ups2
        oalpha_v[s] = alpha
        obeta_v[s] = beta

    pltpu.sync_copy(omu_v, mu_out.at[pl.ds(base, BPW)])
    pltpu.sync_copy(oups_v, ups_out.at[pl.ds(base, BPW)])
    pltpu.sync_copy(oalpha_v, alpha_out.at[pl.ds(base, BPW)])
    pltpu.sync_copy(obeta_v, beta_out.at[pl.ds(base, BPW)])


@jax.jit
def kernel(uid, iid, mu, upsilon, uid_mu_emb, iid_mu_emb,
           uid_upsilon_emb, iid_upsilon_emb):
    mesh = plsc.VectorSubcoreMesh(core_axis_name="c", subcore_axis_name="s")
    f32 = jnp.float32
    i32 = jnp.int32
    uid = uid.astype(i32)
    iid = iid.astype(i32)

    gather_mu = pl.kernel(
        _gather_mu_body,
        mesh=mesh,
        out_type=[jax.ShapeDtypeStruct((B,), f32) for _ in range(2)],
        scratch_types=[
            pltpu.VMEM((BPW,), i32),
            pltpu.VMEM((BPW,), i32),
            pltpu.VMEM((BPW,), f32),
            pltpu.VMEM((BPW,), f32),
            pltpu.SemaphoreType.DMA,
            pltpu.SemaphoreType.DMA,
        ],
    )
    g_umu, g_imu = gather_mu(
        uid, iid,
        uid_mu_emb.reshape(-1),
        iid_mu_emb.reshape(-1),
    )

    ups_math = pl.kernel(
        _ups_math_body,
        mesh=mesh,
        out_type=[jax.ShapeDtypeStruct((B,), f32) for _ in range(4)],
        scratch_types=[
            pltpu.VMEM((BPW,), i32),
            pltpu.VMEM((BPW,), i32),
            pltpu.VMEM((BPW,), f32),
            pltpu.VMEM((BPW,), f32),
            pltpu.VMEM((BPW,), f32),
            pltpu.VMEM((BPW,), f32),
            pltpu.VMEM((BPW,), f32),
            pltpu.VMEM((BPW,), f32),
            pltpu.VMEM((BPW,), f32),
            pltpu.VMEM((BPW,), f32),
            pltpu.VMEM((BPW,), f32),
            pltpu.VMEM((BPW,), f32),
            pltpu.SemaphoreType.DMA,
            pltpu.SemaphoreType.DMA,
        ],
    )
    mu_o, ups_o, al_o, be_o = ups_math(
        uid, iid,
        mu.reshape(B),
        upsilon.reshape(B),
        uid_upsilon_emb.reshape(-1),
        iid_upsilon_emb.reshape(-1),
        g_umu, g_imu,
    )
    return (mu_o.reshape(B, 1), ups_o.reshape(B, 1),
            al_o.reshape(B, 1), be_o.reshape(B, 1))


# speedup vs baseline: 1.0058x; 1.0058x over previous
import functools

import jax
import jax.numpy as jnp
from jax import lax
from jax.experimental import pallas as pl
from jax.experimental.pallas import tpu as pltpu
from jax.experimental.pallas import tpu_sc as plsc

B = 16384
NC = 2
NS = 16
NW = NC * NS
BPW = B // NW
L = 16


def _sc_body(uid_hbm, iid_hbm, mu_hbm, ups_hbm,
             umu_hbm, imu_hbm, uups_hbm, iups_hbm,
             mu_out, ups_out, alpha_out, beta_out,
             uid_v, iid_v, mu_v, ups_v,
             umu_v, imu_v, uups_v, iups_v,
             omu_v, oups_v, oalpha_v, obeta_v,
             sem0, sem1, sem2, sem3):
    wid = lax.axis_index("s") * NC + lax.axis_index("c")
    base = wid * BPW

    pltpu.sync_copy(uid_hbm.at[pl.ds(base, BPW)], uid_v)
    pltpu.sync_copy(iid_hbm.at[pl.ds(base, BPW)], iid_v)

    c0 = pltpu.async_copy(umu_hbm.at[uid_v], umu_v, sem0)
    c1 = pltpu.async_copy(imu_hbm.at[iid_v], imu_v, sem1)
    c2 = pltpu.async_copy(uups_hbm.at[uid_v], uups_v, sem2)
    c3 = pltpu.async_copy(iups_hbm.at[iid_v], iups_v, sem3)

    pltpu.sync_copy(mu_hbm.at[pl.ds(base, BPW)], mu_v)
    pltpu.sync_copy(ups_hbm.at[pl.ds(base, BPW)], ups_v)
    c0.wait()
    c1.wait()
    c2.wait()
    c3.wait()

    eps = jnp.float32(1e-6)
    eps_param = jnp.float32(0.01)
    half = jnp.float32(0.5)
    one = jnp.float32(1.0)
    hi = jnp.float32(15.0)
    zero = jnp.float32(0.0)

    for i in range(BPW // L):
        s = pl.ds(i * L, L)
        mu = mu_v[s]
        ups = ups_v[s]
        umu = umu_v[s]
        imu = imu_v[s]
        uups = uups_v[s]
        iups = iups_v[s]

        ups2 = jnp.minimum(jnp.maximum(ups * (uups * iups), eps), hi)
        prod = umu * imu
        lt = mu < prod
        num = jnp.where(lt, half * mu, half * (mu - prod))
        den = jnp.where(lt, jnp.maximum(prod, eps),
                        jnp.maximum(one - prod, eps))
        off = jnp.where(lt, zero, half)
        mu2 = off + num / den
        alpha = jnp.maximum(mu2 * ups2, eps_param)
        beta = jnp.maximum(ups2 - alpha, eps_param)

        omu_v[s] = mu2
        oups_v[s] = ups2
        oalpha_v[s] = alpha
        obeta_v[s] = beta

    pltpu.sync_copy(omu_v, mu_out.at[pl.ds(base, BPW)])
    pltpu.sync_copy(oups_v, ups_out.at[pl.ds(base, BPW)])
    pltpu.sync_copy(oalpha_v, alpha_out.at[pl.ds(base, BPW)])
    pltpu.sync_copy(obeta_v, beta_out.at[pl.ds(base, BPW)])


@jax.jit
def kernel(uid, iid, mu, upsilon, uid_mu_emb, iid_mu_emb,
           uid_upsilon_emb, iid_upsilon_emb):
    mesh = plsc.VectorSubcoreMesh(core_axis_name="c", subcore_axis_name="s")
    f32 = jnp.float32
    run = pl.kernel(
        _sc_body,
        mesh=mesh,
        out_type=[jax.ShapeDtypeStruct((B,), f32) for _ in range(4)],
        scratch_types=[
            pltpu.VMEM((BPW,), jnp.int32),
            pltpu.VMEM((BPW,), jnp.int32),
            pltpu.VMEM((BPW,), f32),
            pltpu.VMEM((BPW,), f32),
            pltpu.VMEM((BPW,), f32),
            pltpu.VMEM((BPW,), f32),
            pltpu.VMEM((BPW,), f32),
            pltpu.VMEM((BPW,), f32),
            pltpu.VMEM((BPW,), f32),
            pltpu.VMEM((BPW,), f32),
            pltpu.VMEM((BPW,), f32),
            pltpu.VMEM((BPW,), f32),
            pltpu.SemaphoreType.DMA,
            pltpu.SemaphoreType.DMA,
            pltpu.SemaphoreType.DMA,
            pltpu.SemaphoreType.DMA,
        ],
    )
    mu_o, ups_o, al_o, be_o = run(
        uid.astype(jnp.int32),
        iid.astype(jnp.int32),
        mu.reshape(B),
        upsilon.reshape(B),
        uid_mu_emb.reshape(-1),
        iid_mu_emb.reshape(-1),
        uid_upsilon_emb.reshape(-1),
        iid_upsilon_emb.reshape(-1),
    )
    return (mu_o.reshape(B, 1), ups_o.reshape(B, 1),
            al_o.reshape(B, 1), be_o.reshape(B, 1))
